# baseline (speedup 1.0000x reference)
import functools

import jax
import jax.numpy as jnp
from jax import lax
from jax.experimental import pallas as pl
from jax.experimental.pallas import tpu as pltpu
from jax.experimental.pallas import tpu_sc as plsc

ATOMS = 51
D = 128
N = 4096 * 200
NC, NS, L = 2, 16, 16
NW = NC * NS
B_LOC = N // NW
BLK = 256
NBLK = B_LOC // BLK
TBL_WORDS = ATOMS * D


def _make_kernel():
    mesh = plsc.VectorSubcoreMesh(core_axis_name="c", subcore_axis_name="s")

    @functools.partial(
        pl.kernel,
        mesh=mesh,
        out_type=jax.ShapeDtypeStruct((N * D,), jnp.float32),
        compiler_params=pltpu.CompilerParams(needs_layout_passes=False),
        scratch_types=[
            pltpu.VMEM((B_LOC,), jnp.float32),
            pltpu.VMEM((TBL_WORDS,), jnp.float32),
            pltpu.VMEM((BLK * D,), jnp.float32),
        ],
    )
    def body(x_hbm, tbl_hbm, out_hbm, x_v, tbl_v, ob):
        wid = lax.axis_index("s") * NC + lax.axis_index("c")
        base = wid * B_LOC
        pltpu.sync_copy(tbl_hbm, tbl_v)
        pltpu.sync_copy(x_hbm.at[pl.ds(base, B_LOC)], x_v)

        lane = lax.iota(jnp.int32, L)

        def block_body(b, _):
            def group_body(g, _):
                e0 = b * BLK + g * L
                xv = x_v[pl.ds(e0, L)]
                xs = jnp.clip(xv * jnp.float32(ATOMS), 0.0, float(ATOMS - 1))
                fl = xs.astype(jnp.int32)
                d = xs - fl.astype(jnp.float32)
                ce = jnp.minimum(fl + 1, ATOMS - 1)
                ia = fl * D
                ic = ce * D
                io = (g * L + lane) * D

                def feat_body(f, carry):
                    ia, ic, io = carry
                    a = plsc.load_gather(tbl_v, [ia])
                    c = plsc.load_gather(tbl_v, [ic])
                    r = a + d * (c - a)
                    plsc.store_scatter(ob, [io], r)
                    return ia + 1, ic + 1, io + 1

                lax.fori_loop(0, D, feat_body, (ia, ic, io))
                return _

            lax.fori_loop(0, BLK // L, group_body, 0)
            pltpu.sync_copy(ob, out_hbm.at[pl.ds((base + b * BLK) * D, BLK * D)])
            return _

        lax.fori_loop(0, NBLK, block_body, 0)

    return body


_kernel_fn = _make_kernel()


def kernel(x, table):
    xf = x.reshape(N)
    tf = table.reshape(TBL_WORDS)
    out = _kernel_fn(xf, tf)
    return out.reshape(x.shape[0], x.shape[1], D)
